# baseline (speedup 1.0000x reference)
import functools
import math

import jax
import jax.numpy as jnp
from jax import lax
from jax.experimental import pallas as pl
from jax.experimental.pallas import tpu as pltpu
from jax.experimental.pallas import tpu_sc as plsc

D_MODEL = 64
SCALE = math.sqrt(D_MODEL)

NUM_CORES = 2
NUM_SUBCORES = 16
NUM_WORKERS = NUM_CORES * NUM_SUBCORES
LANES = 16

B_TOTAL = 16384 * 50
IDX_ROW = 128
K_PER_STEP = 8
CHUNK = K_PER_STEP * IDX_ROW
ROWS_PER_WORKER = B_TOTAL // NUM_WORKERS
STEPS = ROWS_PER_WORKER // CHUNK
IDX_ROWS_PER_WORKER = ROWS_PER_WORKER // IDX_ROW


def _emb_kernel(idx_hbm, lut_hbm, out_hbm, idx_v, rows_v, sem):
    wid = lax.axis_index("s") * NUM_CORES + lax.axis_index("c")
    idx_row0 = wid * IDX_ROWS_PER_WORKER
    out_row0 = wid * ROWS_PER_WORKER

    def step(i, carry):
        pltpu.sync_copy(idx_hbm.at[pl.ds(idx_row0 + i * K_PER_STEP, K_PER_STEP)],
                        idx_v)
        copies = []
        for j in range(K_PER_STEP):
            copies.append(
                pltpu.async_copy(lut_hbm.at[idx_v.at[j]],
                                 rows_v.at[pl.ds(j * IDX_ROW, IDX_ROW)],
                                 sem))
        for cp in copies:
            cp.wait()

        def scale(r, c):
            for dr in range(4):
                for q in range(4):
                    sl = pl.ds(q * LANES, LANES)
                    rows_v[r * 4 + dr, sl] = rows_v[r * 4 + dr, sl] * SCALE
            return c

        lax.fori_loop(0, CHUNK // 4, scale, 0)

        pltpu.sync_copy(rows_v, out_hbm.at[pl.ds(out_row0 + i * CHUNK, CHUNK)])
        return carry

    lax.fori_loop(0, STEPS, step, 0)


@jax.jit
def kernel(x, lut):
    idx = x.reshape(-1).astype(jnp.int32).reshape(B_TOTAL // IDX_ROW, IDX_ROW)
    mesh = plsc.VectorSubcoreMesh(core_axis_name="c", subcore_axis_name="s")
    run = functools.partial(
        pl.kernel,
        mesh=mesh,
        out_type=jax.ShapeDtypeStruct((B_TOTAL, D_MODEL), jnp.float32),
        scratch_types=[
            pltpu.VMEM((K_PER_STEP, IDX_ROW), jnp.int32),
            pltpu.VMEM((CHUNK, D_MODEL), jnp.float32),
            pltpu.SemaphoreType.DMA,
        ],
        compiler_params=pltpu.CompilerParams(use_tc_tiling_on_sc=False),
    )(_emb_kernel)
    out = run(idx, lut)
    return out.reshape(16384, 50, D_MODEL)


# speedup vs baseline: 1.0594x; 1.0594x over previous
import functools
import math

import jax
import jax.numpy as jnp
from jax import lax
from jax.experimental import pallas as pl
from jax.experimental.pallas import tpu as pltpu
from jax.experimental.pallas import tpu_sc as plsc

D_MODEL = 64
SCALE = math.sqrt(D_MODEL)

NUM_CORES = 2
NUM_SUBCORES = 16
NUM_WORKERS = NUM_CORES * NUM_SUBCORES
LANES = 16

B_TOTAL = 16384 * 50
IDX_ROW = 128
K_PER_STEP = 5
CHUNK = K_PER_STEP * IDX_ROW
ROWS_PER_WORKER = B_TOTAL // NUM_WORKERS
STEPS = ROWS_PER_WORKER // CHUNK
IDX_ROWS_PER_WORKER = ROWS_PER_WORKER // IDX_ROW


def _emb_kernel(idx_hbm, lut_hbm, out_hbm, idx_v, rows0, rows1, gsem0, gsem1,
                osem0, osem1):
    wid = lax.axis_index("s") * NUM_CORES + lax.axis_index("c")
    out_row0 = wid * ROWS_PER_WORKER

    pltpu.sync_copy(idx_hbm.at[pl.ds(wid * IDX_ROWS_PER_WORKER,
                                     IDX_ROWS_PER_WORKER)], idx_v)

    def gathers(buf, sem, s):
        return [
            pltpu.make_async_copy(lut_hbm.at[idx_v.at[s * K_PER_STEP + j]],
                                  buf.at[pl.ds(j * IDX_ROW, IDX_ROW)], sem)
            for j in range(K_PER_STEP)
        ]

    def scatter(buf, sem, s):
        return pltpu.make_async_copy(
            buf, out_hbm.at[pl.ds(out_row0 + s * CHUNK, CHUNK)], sem)

    def scale(buf):
        def body(r, c):
            for dr in range(4):
                for q in range(4):
                    sl = pl.ds(q * LANES, LANES)
                    buf[r * 4 + dr, sl] = buf[r * 4 + dr, sl] * SCALE
            return c

        lax.fori_loop(0, CHUNK // 4, body, 0)

    for cp in gathers(rows0, gsem0, 0):
        cp.start()

    def step(i, carry):
        @pl.when(i > 0)
        def _():
            scatter(rows1, osem1, 2 * i - 1).wait()

        for cp in gathers(rows1, gsem1, 2 * i + 1):
            cp.start()
        for cp in gathers(rows0, gsem0, 2 * i):
            cp.wait()
        scale(rows0)
        scatter(rows0, osem0, 2 * i).start()

        scatter(rows0, osem0, 2 * i).wait()

        @pl.when(i < STEPS // 2 - 1)
        def _():
            for cp in gathers(rows0, gsem0, 2 * i + 2):
                cp.start()

        for cp in gathers(rows1, gsem1, 2 * i + 1):
            cp.wait()
        scale(rows1)
        scatter(rows1, osem1, 2 * i + 1).start()
        return carry

    lax.fori_loop(0, STEPS // 2, step, 0)
    scatter(rows1, osem1, STEPS - 1).wait()


@jax.jit
def kernel(x, lut):
    idx = x.reshape(-1).astype(jnp.int32).reshape(B_TOTAL // IDX_ROW, IDX_ROW)
    mesh = plsc.VectorSubcoreMesh(core_axis_name="c", subcore_axis_name="s")
    run = functools.partial(
        pl.kernel,
        mesh=mesh,
        out_type=jax.ShapeDtypeStruct((B_TOTAL, D_MODEL), jnp.float32),
        scratch_types=[
            pltpu.VMEM((IDX_ROWS_PER_WORKER, IDX_ROW), jnp.int32),
            pltpu.VMEM((CHUNK, D_MODEL), jnp.float32),
            pltpu.VMEM((CHUNK, D_MODEL), jnp.float32),
            pltpu.SemaphoreType.DMA,
            pltpu.SemaphoreType.DMA,
            pltpu.SemaphoreType.DMA,
            pltpu.SemaphoreType.DMA,
        ],
        compiler_params=pltpu.CompilerParams(use_tc_tiling_on_sc=False),
    )(_emb_kernel)
    out = run(idx, lut)
    return out.reshape(16384, 50, D_MODEL)
